# baseline (device time: 151305 ns/iter reference)
import jax
import jax.numpy as jnp
from jax import lax
from jax.experimental import pallas as pl
from jax.experimental.pallas import tpu as pltpu

N_DEV = 16
M = 2048
N = 2048
CHUNK = M // N_DEV
HALF = N // 2
SUB = 4
QCOL = HALF // SUB


def kernel(A, B):
    A16 = A.astype(jnp.bfloat16)
    B16 = B.astype(jnp.bfloat16)

    def body(a_ref, b_ref, out_ref, stage_r, stage_l, ag_r, ag_l,
             send_sems_r, recv_sems_r, send_sems_l, recv_sems_l):
        my = lax.axis_index("i")
        left = lax.rem(my - 1 + N_DEV, N_DEV)
        right = lax.rem(my + 1, N_DEV)

        barrier_sem = pltpu.get_barrier_semaphore()
        for nbr in (left, right):
            pl.semaphore_signal(
                barrier_sem, inc=1,
                device_id=(nbr,), device_id_type=pl.DeviceIdType.MESH,
            )
        pl.semaphore_wait(barrier_sem, 2)

        dirs = [
            dict(sgn=-1, nbr=right, col=0, stage=stage_r, ag=ag_r,
                 ssem=send_sems_r, rsem=recv_sems_r),
            dict(sgn=+1, nbr=left, col=HALF, stage=stage_l, ag=ag_l,
                 ssem=send_sems_l, rsem=recv_sems_l),
        ]

        def idx(k):
            return lax.rem(k + 4 * N_DEV, N_DEV)

        def ptile(r_idx, c0, cw):
            return jnp.dot(
                a_ref[pl.ds(r_idx * CHUNK, CHUNK), :],
                b_ref[:, c0:c0 + cw],
                preferred_element_type=jnp.float32,
            )

        def send(d, j, h, src_buf, src_slice):
            jsl = slice(j * QCOL, (j + 1) * QCOL)
            rdma = pltpu.make_async_remote_copy(
                src_ref=src_buf.at[src_slice, :, jsl],
                dst_ref=d["ag"].at[idx(my + d["sgn"] * h), :, jsl],
                send_sem=d["ssem"].at[h, j],
                recv_sem=d["rsem"].at[h, j],
                device_id=(d["nbr"],),
                device_id_type=pl.DeviceIdType.MESH,
            )
            rdma.start()
            return rdma

        rs = {}
        for d in dirs:
            d["stage"][0] = ptile(idx(my), d["col"], HALF).astype(
                jnp.bfloat16)
            for j in range(SUB):
                rs[(d["sgn"], j)] = [send(d, j, 0, d["stage"], 0)]

        for h in range(N_DEV - 1):
            for j in range(SUB):
                for d in dirs:
                    cj = d["col"] + j * QCOL
                    r_idx = idx(my + d["sgn"] * (h + 1))
                    rd = rs[(d["sgn"], j)]
                    local = ptile(r_idx, cj, QCOL)
                    rd[h].wait_recv()
                    tmp = local + d["ag"][
                        r_idx, :, j * QCOL:(j + 1) * QCOL].astype(jnp.float32)
                    if h < N_DEV - 2:
                        slot = (h + 1) % 2
                        if h >= 1:
                            rd[h - 1].wait_send()
                        d["stage"][slot, :, j * QCOL:(j + 1) * QCOL] = (
                            tmp.astype(jnp.bfloat16))
                        rd.append(send(d, j, h + 1, d["stage"], slot))
                    out_ref[pl.ds(r_idx * CHUNK, CHUNK), cj:cj + QCOL] = tmp
        for rd in rs.values():
            rd[N_DEV - 3].wait_send()
            rd[N_DEV - 2].wait_send()

        def ag_send(d, j, h):
            jsl = slice(j * QCOL, (j + 1) * QCOL)
            s_idx = idx(my - d["sgn"] + d["sgn"] * h)
            rdma = pltpu.make_async_remote_copy(
                src_ref=d["ag"].at[s_idx, :, jsl],
                dst_ref=d["ag"].at[s_idx, :, jsl],
                send_sem=d["ssem"].at[h, j],
                recv_sem=d["rsem"].at[h, j],
                device_id=(d["nbr"],),
                device_id_type=pl.DeviceIdType.MESH,
            )
            rdma.start()
            return rdma

        ag = {}
        for d in dirs:
            own = idx(my - d["sgn"])
            d["ag"][own] = out_ref[
                pl.ds(own * CHUNK, CHUNK), d["col"]:d["col"] + HALF
            ].astype(jnp.bfloat16)
            for j in range(SUB):
                ag[(d["sgn"], j)] = [ag_send(d, j, 0)]

        for h in range(N_DEV - 1):
            for j in range(SUB):
                for d in dirs:
                    cj = d["col"] + j * QCOL
                    r_idx = idx(my + d["sgn"] * h)
                    ag[(d["sgn"], j)][h].wait_recv()
                    if h < N_DEV - 2:
                        ag[(d["sgn"], j)].append(ag_send(d, j, h + 1))
                    out_ref[pl.ds(r_idx * CHUNK, CHUNK), cj:cj + QCOL] = (
                        d["ag"][r_idx, :, j * QCOL:(j + 1) * QCOL].astype(
                            jnp.float32)
                    )
        for rd in ag.values():
            for rdma in rd:
                rdma.wait_send()

    return pl.pallas_call(
        body,
        out_shape=jax.ShapeDtypeStruct((M, N), jnp.float32),
        in_specs=[
            pl.BlockSpec(memory_space=pltpu.VMEM),
            pl.BlockSpec(memory_space=pltpu.VMEM),
        ],
        out_specs=pl.BlockSpec(memory_space=pltpu.VMEM),
        scratch_shapes=[
            pltpu.VMEM((2, CHUNK, HALF), jnp.bfloat16),
            pltpu.VMEM((2, CHUNK, HALF), jnp.bfloat16),
            pltpu.VMEM((N_DEV, CHUNK, HALF), jnp.bfloat16),
            pltpu.VMEM((N_DEV, CHUNK, HALF), jnp.bfloat16),
            pltpu.SemaphoreType.DMA((N_DEV - 1, SUB)),
            pltpu.SemaphoreType.DMA((N_DEV - 1, SUB)),
            pltpu.SemaphoreType.DMA((N_DEV - 1, SUB)),
            pltpu.SemaphoreType.DMA((N_DEV - 1, SUB)),
        ],
        compiler_params=pltpu.CompilerParams(collective_id=0),
    )(A16, B16)


# device time: 142883 ns/iter; 1.0589x vs baseline; 1.0589x over previous
import jax
import jax.numpy as jnp
from jax import lax
from jax.experimental import pallas as pl
from jax.experimental.pallas import tpu as pltpu

N_DEV = 16
M = 2048
N = 2048
CHUNK = M // N_DEV
HALF = N // 2
SUB = 4
QCOL = HALF // SUB


def kernel(A, B):
    partial = jnp.dot(
        A.astype(jnp.bfloat16),
        B.astype(jnp.bfloat16),
        preferred_element_type=jnp.float32,
    )

    def body(p_ref, out_ref, stage_r, stage_l, ag_r, ag_l,
             send_sems_r, recv_sems_r, send_sems_l, recv_sems_l):
        my = lax.axis_index("i")
        left = lax.rem(my - 1 + N_DEV, N_DEV)
        right = lax.rem(my + 1, N_DEV)

        barrier_sem = pltpu.get_barrier_semaphore()
        for nbr in (left, right):
            pl.semaphore_signal(
                barrier_sem, inc=1,
                device_id=(nbr,), device_id_type=pl.DeviceIdType.MESH,
            )
        pl.semaphore_wait(barrier_sem, 2)

        dirs = [
            dict(sgn=-1, nbr=right, col=0, stage=stage_r, ag=ag_r,
                 ssem=send_sems_r, rsem=recv_sems_r),
            dict(sgn=+1, nbr=left, col=HALF, stage=stage_l, ag=ag_l,
                 ssem=send_sems_l, rsem=recv_sems_l),
        ]

        def idx(k):
            return lax.rem(k + 4 * N_DEV, N_DEV)

        def send(d, j, h, src_buf, src_slice):
            jsl = slice(j * QCOL, (j + 1) * QCOL)
            rdma = pltpu.make_async_remote_copy(
                src_ref=src_buf.at[src_slice, :, jsl],
                dst_ref=d["ag"].at[idx(my + d["sgn"] * h), :, jsl],
                send_sem=d["ssem"].at[h, j],
                recv_sem=d["rsem"].at[h, j],
                device_id=(d["nbr"],),
                device_id_type=pl.DeviceIdType.MESH,
            )
            rdma.start()
            return rdma

        rs = {}
        for d in dirs:
            s0 = idx(my)
            d["stage"][0] = p_ref[
                pl.ds(s0 * CHUNK, CHUNK), d["col"]:d["col"] + HALF
            ].astype(jnp.bfloat16)
            for j in range(SUB):
                rs[(d["sgn"], j)] = [send(d, j, 0, d["stage"], 0)]

        for h in range(N_DEV - 1):
            for j in range(SUB):
                for d in dirs:
                    cj = d["col"] + j * QCOL
                    r_idx = idx(my + d["sgn"] * (h + 1))
                    rd = rs[(d["sgn"], j)]
                    rd[h].wait_recv()
                    tmp = (
                        p_ref[pl.ds(r_idx * CHUNK, CHUNK), cj:cj + QCOL]
                        + d["ag"][r_idx, :, j * QCOL:(j + 1) * QCOL].astype(
                            jnp.float32)
                    )
                    if h < N_DEV - 2:
                        slot = (h + 1) % 2
                        if h >= 1:
                            rd[h - 1].wait_send()
                        d["stage"][slot, :, j * QCOL:(j + 1) * QCOL] = (
                            tmp.astype(jnp.bfloat16))
                        rd.append(send(d, j, h + 1, d["stage"], slot))
                    out_ref[pl.ds(r_idx * CHUNK, CHUNK), cj:cj + QCOL] = tmp
        for key, rd in rs.items():
            rd[N_DEV - 3].wait_send()
            rd[N_DEV - 2].wait_send()

        def ag_send(d, j, h):
            jsl = slice(j * QCOL, (j + 1) * QCOL)
            s_idx = idx(my - d["sgn"] + d["sgn"] * h)
            rdma = pltpu.make_async_remote_copy(
                src_ref=d["ag"].at[s_idx, :, jsl],
                dst_ref=d["ag"].at[s_idx, :, jsl],
                send_sem=d["ssem"].at[h, j],
                recv_sem=d["rsem"].at[h, j],
                device_id=(d["nbr"],),
                device_id_type=pl.DeviceIdType.MESH,
            )
            rdma.start()
            return rdma

        ag = {}
        for d in dirs:
            own = idx(my - d["sgn"])
            d["ag"][own] = out_ref[
                pl.ds(own * CHUNK, CHUNK), d["col"]:d["col"] + HALF
            ].astype(jnp.bfloat16)
            for j in range(SUB):
                ag[(d["sgn"], j)] = [ag_send(d, j, 0)]

        for h in range(N_DEV - 1):
            for j in range(SUB):
                for d in dirs:
                    cj = d["col"] + j * QCOL
                    r_idx = idx(my + d["sgn"] * h)
                    ag[(d["sgn"], j)][h].wait_recv()
                    if h < N_DEV - 2:
                        ag[(d["sgn"], j)].append(ag_send(d, j, h + 1))
                    out_ref[pl.ds(r_idx * CHUNK, CHUNK), cj:cj + QCOL] = (
                        d["ag"][r_idx, :, j * QCOL:(j + 1) * QCOL].astype(
                            jnp.float32)
                    )
        for key, rd in ag.items():
            for rdma in rd:
                rdma.wait_send()

    return pl.pallas_call(
        body,
        out_shape=jax.ShapeDtypeStruct((M, N), jnp.float32),
        in_specs=[pl.BlockSpec(memory_space=pltpu.VMEM)],
        out_specs=pl.BlockSpec(memory_space=pltpu.VMEM),
        scratch_shapes=[
            pltpu.VMEM((2, CHUNK, HALF), jnp.bfloat16),
            pltpu.VMEM((2, CHUNK, HALF), jnp.bfloat16),
            pltpu.VMEM((N_DEV, CHUNK, HALF), jnp.bfloat16),
            pltpu.VMEM((N_DEV, CHUNK, HALF), jnp.bfloat16),
            pltpu.SemaphoreType.DMA((N_DEV - 1, SUB)),
            pltpu.SemaphoreType.DMA((N_DEV - 1, SUB)),
            pltpu.SemaphoreType.DMA((N_DEV - 1, SUB)),
            pltpu.SemaphoreType.DMA((N_DEV - 1, SUB)),
        ],
        compiler_params=pltpu.CompilerParams(collective_id=0),
    )(partial)
